# Initial kernel scaffold; baseline (speedup 1.0000x reference)
import jax
import jax.numpy as jnp
from jax.experimental import pallas as pl


def kernel(x, edge_index, batch, W1, b1, W2, b2, W3, b3, Wc1, bc1, Wc2, bc2):
    raise NotImplementedError("write your pallas kernel here")


# speedup vs baseline: 3.8611x; 3.8611x over previous
import functools

import jax
import jax.numpy as jnp
from jax import lax
from jax.experimental import pallas as pl
from jax.experimental.pallas import tpu as pltpu
from jax.experimental.pallas import tpu_sc as plsc

N = 10000
E = 320000
F_IN = 128
H = 64
G = 16
NUM_CLASSES = 2

NC = 2
NS = 16
NT = NC * NS
RPT = 320
NPAD = NT * RPT
CAP = 24576
BATCH = 128
CHUNK = 6400
TRASH = RPT
_BIS = 5


def _mesh():
    return plsc.VectorSubcoreMesh(core_axis_name="c", subcore_axis_name="s")


def _wid():
    return lax.axis_index("s") * NC + lax.axis_index("c")


def _rsqrt_newton(x):
    i = plsc.bitcast(x, jnp.int32)
    i = jnp.int32(0x5F3759DF) - lax.shift_right_arithmetic(i, 1)
    y = plsc.bitcast(i, jnp.float32)
    for _ in range(3):
        y = y * (1.5 - 0.5 * x * y * y)
    return y


def _build_prep():
    def body(edge_hbm, srcl_hbm, dstl_hbm, cnt_hbm, dinv_hbm,
             srcb, dstb, srcl, dstl, deg, dinvb, cntb):
        wid = _wid()
        lo = wid * RPT
        iota = lax.iota(jnp.int32, 16)
        ones = jnp.ones((16,), jnp.float32)

        for g in range(RPT // 16):
            deg[pl.ds(g * 16, 16)] = jnp.zeros((16,), jnp.float32)

        def chunk_step(c, cntv):
            pltpu.sync_copy(edge_hbm.at[0, pl.ds(c * CHUNK, CHUNK)], srcb)
            pltpu.sync_copy(edge_hbm.at[1, pl.ds(c * CHUNK, CHUNK)], dstb)

            def grp(g, cv):
                s16 = srcb[pl.ds(g * 16, 16)]
                d16 = dstb[pl.ds(g * 16, 16)]
                dl = d16 - lo
                m = (dl >= 0) & (dl < RPT)
                if _BIS >= 4:
                    mi = m.astype(jnp.int32)
                    pos = cv + plsc.cumsum(mi) - 1
                    plsc.store_scatter(srcl, [pos], s16, mask=m)
                    plsc.store_scatter(dstl, [pos], dl, mask=m)
                if _BIS >= 5:
                    plsc.addupdate_scatter(deg, [dl], ones, mask=m)
                return cv + plsc.all_reduce_population_count(m)

            cntv = lax.fori_loop(0, CHUNK // 16, grp, cntv)
            return jnp.minimum(cntv, jnp.int32(CAP - CHUNK - BATCH - 16))

        if _BIS >= 3:
            cntv = lax.fori_loop(0, E // CHUNK, chunk_step,
                                 jnp.zeros((16,), jnp.int32))
        else:
            cntv = jnp.zeros((16,), jnp.int32)

        if _BIS >= 2:
            for g in range(BATCH // 16):
                pos = cntv + (g * 16 + iota)
                plsc.store_scatter(srcl, [pos], jnp.zeros((16,), jnp.int32))
                plsc.store_scatter(dstl, [pos],
                                   jnp.full((16,), TRASH, jnp.int32))
        cntp = jnp.bitwise_and(cntv + (BATCH - 1), jnp.int32(-BATCH))
        cntb[...] = cntp

        for g in range(RPT // 16):
            node = lo + g * 16 + iota
            dg = deg[pl.ds(g * 16, 16)] + 1.0
            y = _rsqrt_newton(dg)
            dinvb[pl.ds(g * 16, 16)] = jnp.where(node < N, y, 0.0)

        pltpu.sync_copy(dinvb, dinv_hbm.at[pl.ds(lo, RPT)])
        pltpu.sync_copy(cntb, cnt_hbm.at[wid])
        pltpu.sync_copy(srcl, srcl_hbm.at[wid])
        pltpu.sync_copy(dstl, dstl_hbm.at[wid])

    return pl.kernel(
        body,
        out_type=(
            jax.ShapeDtypeStruct((NT, CAP), jnp.int32),
            jax.ShapeDtypeStruct((NT, CAP), jnp.int32),
            jax.ShapeDtypeStruct((NT, 16), jnp.int32),
            jax.ShapeDtypeStruct((NPAD,), jnp.float32),
        ),
        mesh=_mesh(),
        compiler_params=pltpu.CompilerParams(needs_layout_passes=False, use_tc_tiling_on_sc=False),
        scratch_types=[
            pltpu.VMEM((CHUNK,), jnp.int32),
            pltpu.VMEM((CHUNK,), jnp.int32),
            pltpu.VMEM((CAP,), jnp.int32),
            pltpu.VMEM((CAP,), jnp.int32),
            pltpu.VMEM((RPT,), jnp.float32),
            pltpu.VMEM((RPT,), jnp.float32),
            pltpu.VMEM((16,), jnp.int32),
        ],
    )


def _build_spmm(width):
    def body(hws_hbm, srcl_hbm, dstl_hbm, cnt_hbm, out_hbm,
             srcb, dstb, rows, acc, cntb, semg):
        wid = _wid()
        lo = wid * RPT
        iota = lax.iota(jnp.int32, 16)

        pltpu.sync_copy(hws_hbm.at[pl.ds(lo, RPT)], acc.at[pl.ds(0, RPT)])
        zf = jnp.zeros((16,), jnp.float32)
        trow = jnp.full((16,), TRASH, jnp.int32)
        for j in range(width // 16):
            plsc.store_scatter(acc, [trow, j * 16 + iota], zf)

        pltpu.sync_copy(cnt_hbm.at[wid], cntb)
        nb = jnp.max(cntb[...]) // BATCH

        def batch_step(b, carry):
            pltpu.sync_copy(srcl_hbm.at[wid, pl.ds(b * BATCH, BATCH)], srcb)
            pltpu.sync_copy(dstl_hbm.at[wid, pl.ds(b * BATCH, BATCH)], dstb)
            pltpu.async_copy(hws_hbm.at[srcb], rows, semg).wait()
            d16s = [dstb[pl.ds(g * 16, 16)] for g in range(BATCH // 16)]
            rids = [g * 16 + iota for g in range(BATCH // 16)]

            def jstep(j, js):
                for g in range(BATCH // 16):
                    col = plsc.load_gather(rows, [rids[g], js])
                    plsc.addupdate_scatter(acc, [d16s[g], js], col)
                return js + 1

            lax.fori_loop(0, width, jstep, jnp.zeros((16,), jnp.int32))
            return carry

        lax.fori_loop(0, nb, batch_step, jnp.int32(0))
        pltpu.sync_copy(acc.at[pl.ds(0, RPT)], out_hbm.at[pl.ds(lo, RPT)])

    return pl.kernel(
        body,
        out_type=jax.ShapeDtypeStruct((NPAD, width), jnp.float32),
        mesh=_mesh(),
        compiler_params=pltpu.CompilerParams(needs_layout_passes=False, use_tc_tiling_on_sc=False),
        scratch_types=[
            pltpu.VMEM((BATCH,), jnp.int32),
            pltpu.VMEM((BATCH,), jnp.int32),
            pltpu.VMEM((BATCH, width), jnp.float32),
            pltpu.VMEM((RPT + 1, width), jnp.float32),
            pltpu.VMEM((16,), jnp.int32),
            pltpu.SemaphoreType.DMA,
        ],
    )


_BLK = 512


def _mm_first(x, w, dinv_col):
    def body(x_ref, w_ref, dv_ref, o_ref):
        o_ref[...] = jnp.dot(x_ref[...], w_ref[...],
                             preferred_element_type=jnp.float32) * dv_ref[...]

    kin, kout = w.shape
    return pl.pallas_call(
        body,
        grid=(NPAD // _BLK,),
        in_specs=[
            pl.BlockSpec((_BLK, kin), lambda i: (i, 0)),
            pl.BlockSpec((kin, kout), lambda i: (0, 0)),
            pl.BlockSpec((_BLK, 1), lambda i: (i, 0)),
        ],
        out_specs=pl.BlockSpec((_BLK, kout), lambda i: (i, 0)),
        out_shape=jax.ShapeDtypeStruct((NPAD, kout), jnp.float32),
    )(x, w, dinv_col)


def _mm_mid(y, w, dinv_col, b_row):
    def body(y_ref, w_ref, dv_ref, b_ref, o_ref):
        h = jnp.maximum(y_ref[...] * dv_ref[...] + b_ref[...], 0.0)
        o_ref[...] = jnp.dot(h, w_ref[...],
                             preferred_element_type=jnp.float32) * dv_ref[...]

    kin, kout = w.shape
    return pl.pallas_call(
        body,
        grid=(NPAD // _BLK,),
        in_specs=[
            pl.BlockSpec((_BLK, kin), lambda i: (i, 0)),
            pl.BlockSpec((kin, kout), lambda i: (0, 0)),
            pl.BlockSpec((_BLK, 1), lambda i: (i, 0)),
            pl.BlockSpec((1, kin), lambda i: (0, 0)),
        ],
        out_specs=pl.BlockSpec((_BLK, kout), lambda i: (i, 0)),
        out_shape=jax.ShapeDtypeStruct((NPAD, kout), jnp.float32),
    )(y, w, dinv_col, b_row)


_BLKF = 1024


def _pool_head(y3, dinv_col, batch2d, b3_row, wc1, bc1_row, wc2, bc2_row):
    w3 = y3.shape[1]
    nblk = NPAD // _BLKF

    def body(y_ref, dv_ref, bat_ref, b3_ref, wc1_ref, bc1_ref, wc2_ref,
             bc2_ref, o_ref, psum, pmax):
        i = pl.program_id(0)

        @pl.when(i == 0)
        def _():
            psum[...] = jnp.zeros((G, w3), jnp.float32)
            pmax[...] = jnp.full((G, w3), -jnp.inf, jnp.float32)

        h = y_ref[...] * dv_ref[...]
        bat = bat_ref[...]
        rowi = (lax.broadcasted_iota(jnp.int32, (_BLKF, w3), 0) + i * _BLKF)
        starts = [jnp.sum((bat < g).astype(jnp.int32)) for g in range(G + 1)]
        for g in range(G):
            sg, eg = starts[g], starts[g + 1]
            m = (rowi >= sg) & (rowi < eg)
            sumg = jnp.sum(jnp.where(m, h, 0.0), axis=0, keepdims=True)
            maxg = jnp.max(jnp.where(m, h, -jnp.inf), axis=0, keepdims=True)
            psum[pl.ds(g, 1), :] += sumg
            pmax[pl.ds(g, 1), :] = jnp.maximum(pmax[pl.ds(g, 1), :], maxg)

        @pl.when(i == nblk - 1)
        def _():
            b3 = b3_ref[...]
            cnts = jnp.concatenate(
                [jnp.reshape((starts[g + 1] - starts[g]).astype(jnp.float32),
                             (1, 1)) for g in range(G)], axis=0)
            means = (psum[...] + cnts * b3) / jnp.maximum(cnts, 1.0)
            maxs = pmax[...] + b3
            z = jnp.concatenate([means, maxs], axis=1)
            r = jnp.maximum(jnp.dot(z, wc1_ref[...],
                                    preferred_element_type=jnp.float32)
                            + bc1_ref[...], 0.0)
            o_ref[...] = jnp.dot(r, wc2_ref[...],
                                 preferred_element_type=jnp.float32) \
                + bc2_ref[...]

    return pl.pallas_call(
        body,
        grid=(nblk,),
        in_specs=[
            pl.BlockSpec((_BLKF, w3), lambda i: (i, 0)),
            pl.BlockSpec((_BLKF, 1), lambda i: (i, 0)),
            pl.BlockSpec(batch2d.shape, lambda i: (0, 0)),
            pl.BlockSpec(b3_row.shape, lambda i: (0, 0)),
            pl.BlockSpec(wc1.shape, lambda i: (0, 0)),
            pl.BlockSpec(bc1_row.shape, lambda i: (0, 0)),
            pl.BlockSpec(wc2.shape, lambda i: (0, 0)),
            pl.BlockSpec(bc2_row.shape, lambda i: (0, 0)),
        ],
        out_specs=pl.BlockSpec((G, NUM_CLASSES), lambda i: (0, 0)),
        out_shape=jax.ShapeDtypeStruct((G, NUM_CLASSES), jnp.float32),
        scratch_shapes=[
            pltpu.VMEM((G, w3), jnp.float32),
            pltpu.VMEM((G, w3), jnp.float32),
        ],
    )(y3, dinv_col, batch2d, b3_row, wc1, bc1_row, wc2, bc2_row)


def kernel(x, edge_index, batch, W1, b1, W2, b2, W3, b3, Wc1, bc1, Wc2, bc2):
    prep = _build_prep()
    spmm64 = _build_spmm(H)
    spmm32 = _build_spmm(H // 2)

    srcl, dstl, cnt, dinv = prep(edge_index)
    dinv_col = dinv.reshape(NPAD, 1)

    x_pad = jnp.pad(x, ((0, NPAD - N), (0, 0)))
    batch2d = jnp.pad(batch, (0, NPAD - N), constant_values=G).reshape(
        NPAD // 128, 128)

    hws1 = _mm_first(x_pad, W1, dinv_col)
    y1 = spmm64(hws1, srcl, dstl, cnt)
    hws2 = _mm_mid(y1, W2, dinv_col, b1.reshape(1, H))
    y2 = spmm64(hws2, srcl, dstl, cnt)
    hws3 = _mm_mid(y2, W3, dinv_col, b2.reshape(1, H))
    y3 = spmm32(hws3, srcl, dstl, cnt)
    return _pool_head(y3, dinv_col, batch2d, b3.reshape(1, H // 2),
                      Wc1, bc1.reshape(1, H), Wc2,
                      bc2.reshape(1, NUM_CLASSES))


# speedup vs baseline: 4.0286x; 1.0434x over previous
import functools

import jax
import jax.numpy as jnp
from jax import lax
from jax.experimental import pallas as pl
from jax.experimental.pallas import tpu as pltpu
from jax.experimental.pallas import tpu_sc as plsc

N = 10000
E = 320000
F_IN = 128
H = 64
G = 16
NUM_CLASSES = 2

NC = 2
NS = 16
NT = NC * NS
RPT = 320
NPAD = NT * RPT
CAP = 24576
BATCH = 128
CHUNK = 6400
TRASH = RPT
_BIS = 5


def _mesh():
    return plsc.VectorSubcoreMesh(core_axis_name="c", subcore_axis_name="s")


def _wid():
    return lax.axis_index("s") * NC + lax.axis_index("c")


def _rsqrt_newton(x):
    i = plsc.bitcast(x, jnp.int32)
    i = jnp.int32(0x5F3759DF) - lax.shift_right_arithmetic(i, 1)
    y = plsc.bitcast(i, jnp.float32)
    for _ in range(3):
        y = y * (1.5 - 0.5 * x * y * y)
    return y


NBATCH = CAP // BATCH


def _build_prep():
    nchunks = E // CHUNK

    def body(edge_hbm, list_hbm, cnt_hbm, dinv_hbm,
             ebuf, l2, deg, dinvb, cntb, sem0, sem1):
        wid = _wid()
        lo = wid * RPT
        iota = lax.iota(jnp.int32, 16)
        ones = jnp.ones((16,), jnp.float32)
        zeros_i = jnp.zeros((16,), jnp.int32)
        ones_i = jnp.ones((16,), jnp.int32)
        sems = [sem0, sem1]

        for g in range(RPT // 16):
            deg[pl.ds(g * 16, 16)] = jnp.zeros((16,), jnp.float32)

        def issue(c, s):
            pltpu.async_copy(edge_hbm.at[:, pl.ds(c * CHUNK, CHUNK)],
                             ebuf.at[s], sems[s])

        def wait(s):
            pltpu.make_async_copy(edge_hbm.at[:, pl.ds(0, CHUNK)],
                                  ebuf.at[s], sems[s]).wait()

        issue(0, 0)
        issue(1, 1)

        def scat(pos, s16, dl16, m):
            hi7 = lax.shift_right_logical(pos, 7)
            lo7 = jnp.bitwise_and(pos, 127)
            plsc.store_scatter(l2, [hi7, zeros_i, lo7], s16, mask=m)
            plsc.store_scatter(l2, [hi7, ones_i, lo7], dl16, mask=m)

        cntv = jnp.zeros((16,), jnp.int32)
        for c in range(nchunks):
            s = c % 2
            wait(s)

            def grp(g, cv):
                s16 = ebuf[s, 0, pl.ds(g * 16, 16)]
                d16 = ebuf[s, 1, pl.ds(g * 16, 16)]
                dl = d16 - lo
                m = (dl >= 0) & (dl < RPT)
                pos = cv + plsc.cumsum(m.astype(jnp.int32)) - 1
                scat(pos, s16, dl, m)
                plsc.addupdate_scatter(deg, [dl], ones, mask=m)
                return cv + plsc.all_reduce_population_count(m)

            cntv = lax.fori_loop(0, CHUNK // 16, grp, cntv)
            cntv = jnp.minimum(cntv, jnp.int32(CAP - CHUNK - 5 * BATCH))
            if c + 2 < nchunks:
                issue(c + 2, s)

        for g in range(4 * BATCH // 16):
            pos = cntv + (g * 16 + iota)
            scat(pos, zeros_i, jnp.full((16,), TRASH, jnp.int32),
                 jnp.ones((16,), jnp.bool_))
        cntp = jnp.bitwise_and(cntv + (2 * BATCH - 1), jnp.int32(-2 * BATCH))
        cntb[...] = cntp

        for g in range(RPT // 16):
            node = lo + g * 16 + iota
            dg = deg[pl.ds(g * 16, 16)] + 1.0
            y = _rsqrt_newton(dg)
            dinvb[pl.ds(g * 16, 16)] = jnp.where(node < N, y, 0.0)

        pltpu.sync_copy(dinvb, dinv_hbm.at[pl.ds(lo, RPT)])
        pltpu.sync_copy(cntb, cnt_hbm.at[wid])
        pltpu.sync_copy(l2, list_hbm.at[wid])

    return pl.kernel(
        body,
        out_type=(
            jax.ShapeDtypeStruct((NT, NBATCH, 2, BATCH), jnp.int32),
            jax.ShapeDtypeStruct((NT, 16), jnp.int32),
            jax.ShapeDtypeStruct((NPAD,), jnp.float32),
        ),
        mesh=_mesh(),
        compiler_params=pltpu.CompilerParams(needs_layout_passes=False,
                                             use_tc_tiling_on_sc=False),
        scratch_types=[
            pltpu.VMEM((2, 2, CHUNK), jnp.int32),
            pltpu.VMEM((NBATCH, 2, BATCH), jnp.int32),
            pltpu.VMEM((RPT,), jnp.float32),
            pltpu.VMEM((RPT,), jnp.float32),
            pltpu.VMEM((16,), jnp.int32),
            pltpu.SemaphoreType.DMA,
            pltpu.SemaphoreType.DMA,
        ],
    )


def _build_spmm(width):
    def body(hws_hbm, list_hbm, cnt_hbm, out_hbm,
             idxb, rows, acc, cntb, semi0, semi1, semg0, semg1):
        wid = _wid()
        lo = wid * RPT
        iota = lax.iota(jnp.int32, 16)
        semi = [semi0, semi1]
        semg = [semg0, semg1]

        pltpu.sync_copy(hws_hbm.at[pl.ds(lo, RPT)], acc.at[pl.ds(0, RPT)])
        zf = jnp.zeros((16,), jnp.float32)
        trow = jnp.full((16,), TRASH, jnp.int32)
        for j in range(width // 16):
            plsc.store_scatter(acc, [trow, j * 16 + iota], zf)

        pltpu.sync_copy(cnt_hbm.at[wid], cntb)
        nb2 = jnp.max(cntb[...]) // (2 * BATCH)

        def issue_idx(b, s):
            pltpu.async_copy(list_hbm.at[wid, b], idxb.at[s], semi[s])

        def wait_idx(s):
            pltpu.make_async_copy(list_hbm.at[wid, 0], idxb.at[s],
                                  semi[s]).wait()

        def issue_gather(s):
            pltpu.async_copy(hws_hbm.at[idxb.at[s, 0]], rows.at[s], semg[s])

        def wait_gather(s):
            pltpu.make_async_copy(hws_hbm.at[pl.ds(0, BATCH)], rows.at[s],
                                  semg[s]).wait()

        rids = [g * 16 + iota for g in range(BATCH // 16)]

        def accumulate(s):
            d16s = [idxb[s, 1, pl.ds(g * 16, 16)]
                    for g in range(BATCH // 16)]

            def jstep(j, js):
                for g in range(BATCH // 16):
                    col = plsc.load_gather(rows.at[s], [rids[g], js])
                    plsc.addupdate_scatter(acc, [d16s[g], js], col)
                return js + 1

            lax.fori_loop(0, width, jstep, jnp.zeros((16,), jnp.int32))

        issue_idx(0, 0)
        wait_idx(0)
        issue_gather(0)
        issue_idx(1, 1)

        def pair_step(i, carry):
            for q in (0, 1):
                b = 2 * i + q
                other = 1 - q
                wait_idx(other)
                issue_gather(other)
                wait_gather(q)
                accumulate(q)
                issue_idx(b + 2, q)
            return carry

        lax.fori_loop(0, nb2, pair_step, jnp.int32(0))
        wait_gather(0)
        wait_idx(1)

        pltpu.sync_copy(acc.at[pl.ds(0, RPT)], out_hbm.at[pl.ds(lo, RPT)])

    return pl.kernel(
        body,
        out_type=jax.ShapeDtypeStruct((NPAD, width), jnp.float32),
        mesh=_mesh(),
        compiler_params=pltpu.CompilerParams(needs_layout_passes=False,
                                             use_tc_tiling_on_sc=False),
        scratch_types=[
            pltpu.VMEM((2, 2, BATCH), jnp.int32),
            pltpu.VMEM((2, BATCH, width), jnp.float32),
            pltpu.VMEM((RPT + 1, width), jnp.float32),
            pltpu.VMEM((16,), jnp.int32),
            pltpu.SemaphoreType.DMA,
            pltpu.SemaphoreType.DMA,
            pltpu.SemaphoreType.DMA,
            pltpu.SemaphoreType.DMA,
        ],
    )


_BLK = 512


def _mm_first(x, w, dinv_col):
    def body(x_ref, w_ref, dv_ref, o_ref):
        o_ref[...] = jnp.dot(x_ref[...], w_ref[...],
                             preferred_element_type=jnp.float32) * dv_ref[...]

    kin, kout = w.shape
    return pl.pallas_call(
        body,
        grid=(NPAD // _BLK,),
        in_specs=[
            pl.BlockSpec((_BLK, kin), lambda i: (i, 0)),
            pl.BlockSpec((kin, kout), lambda i: (0, 0)),
            pl.BlockSpec((_BLK, 1), lambda i: (i, 0)),
        ],
        out_specs=pl.BlockSpec((_BLK, kout), lambda i: (i, 0)),
        out_shape=jax.ShapeDtypeStruct((NPAD, kout), jnp.float32),
    )(x, w, dinv_col)


def _mm_mid(y, w, dinv_col, b_row):
    def body(y_ref, w_ref, dv_ref, b_ref, o_ref):
        h = jnp.maximum(y_ref[...] * dv_ref[...] + b_ref[...], 0.0)
        o_ref[...] = jnp.dot(h, w_ref[...],
                             preferred_element_type=jnp.float32) * dv_ref[...]

    kin, kout = w.shape
    return pl.pallas_call(
        body,
        grid=(NPAD // _BLK,),
        in_specs=[
            pl.BlockSpec((_BLK, kin), lambda i: (i, 0)),
            pl.BlockSpec((kin, kout), lambda i: (0, 0)),
            pl.BlockSpec((_BLK, 1), lambda i: (i, 0)),
            pl.BlockSpec((1, kin), lambda i: (0, 0)),
        ],
        out_specs=pl.BlockSpec((_BLK, kout), lambda i: (i, 0)),
        out_shape=jax.ShapeDtypeStruct((NPAD, kout), jnp.float32),
    )(y, w, dinv_col, b_row)


_BLKF = 1024


def _pool_head(y3, dinv_col, batch2d, b3_row, wc1, bc1_row, wc2, bc2_row):
    w3 = y3.shape[1]
    nblk = NPAD // _BLKF

    def body(y_ref, dv_ref, bat_ref, b3_ref, wc1_ref, bc1_ref, wc2_ref,
             bc2_ref, o_ref, psum, pmax):
        i = pl.program_id(0)

        @pl.when(i == 0)
        def _():
            psum[...] = jnp.zeros((G, w3), jnp.float32)
            pmax[...] = jnp.full((G, w3), -jnp.inf, jnp.float32)

        h = y_ref[...] * dv_ref[...]
        bat = bat_ref[...]
        rowi = (lax.broadcasted_iota(jnp.int32, (_BLKF, w3), 0) + i * _BLKF)
        starts = [jnp.sum((bat < g).astype(jnp.int32)) for g in range(G + 1)]
        for g in range(G):
            sg, eg = starts[g], starts[g + 1]
            m = (rowi >= sg) & (rowi < eg)
            sumg = jnp.sum(jnp.where(m, h, 0.0), axis=0, keepdims=True)
            maxg = jnp.max(jnp.where(m, h, -jnp.inf), axis=0, keepdims=True)
            psum[pl.ds(g, 1), :] += sumg
            pmax[pl.ds(g, 1), :] = jnp.maximum(pmax[pl.ds(g, 1), :], maxg)

        @pl.when(i == nblk - 1)
        def _():
            b3 = b3_ref[...]
            cnts = jnp.concatenate(
                [jnp.reshape((starts[g + 1] - starts[g]).astype(jnp.float32),
                             (1, 1)) for g in range(G)], axis=0)
            means = (psum[...] + cnts * b3) / jnp.maximum(cnts, 1.0)
            maxs = pmax[...] + b3
            z = jnp.concatenate([means, maxs], axis=1)
            r = jnp.maximum(jnp.dot(z, wc1_ref[...],
                                    preferred_element_type=jnp.float32)
                            + bc1_ref[...], 0.0)
            o_ref[...] = jnp.dot(r, wc2_ref[...],
                                 preferred_element_type=jnp.float32) \
                + bc2_ref[...]

    return pl.pallas_call(
        body,
        grid=(nblk,),
        in_specs=[
            pl.BlockSpec((_BLKF, w3), lambda i: (i, 0)),
            pl.BlockSpec((_BLKF, 1), lambda i: (i, 0)),
            pl.BlockSpec(batch2d.shape, lambda i: (0, 0)),
            pl.BlockSpec(b3_row.shape, lambda i: (0, 0)),
            pl.BlockSpec(wc1.shape, lambda i: (0, 0)),
            pl.BlockSpec(bc1_row.shape, lambda i: (0, 0)),
            pl.BlockSpec(wc2.shape, lambda i: (0, 0)),
            pl.BlockSpec(bc2_row.shape, lambda i: (0, 0)),
        ],
        out_specs=pl.BlockSpec((G, NUM_CLASSES), lambda i: (0, 0)),
        out_shape=jax.ShapeDtypeStruct((G, NUM_CLASSES), jnp.float32),
        scratch_shapes=[
            pltpu.VMEM((G, w3), jnp.float32),
            pltpu.VMEM((G, w3), jnp.float32),
        ],
    )(y3, dinv_col, batch2d, b3_row, wc1, bc1_row, wc2, bc2_row)


def kernel(x, edge_index, batch, W1, b1, W2, b2, W3, b3, Wc1, bc1, Wc2, bc2):
    prep = _build_prep()
    spmm64 = _build_spmm(H)
    spmm32 = _build_spmm(H // 2)

    lists, cnt, dinv = prep(edge_index)
    dinv_col = dinv.reshape(NPAD, 1)

    x_pad = jnp.pad(x, ((0, NPAD - N), (0, 0)))
    batch2d = jnp.pad(batch, (0, NPAD - N), constant_values=G).reshape(
        NPAD // 128, 128)

    hws1 = _mm_first(x_pad, W1, dinv_col)
    y1 = spmm64(hws1, lists, cnt)
    hws2 = _mm_mid(y1, W2, dinv_col, b1.reshape(1, H))
    y2 = spmm64(hws2, lists, cnt)
    hws3 = _mm_mid(y2, W3, dinv_col, b2.reshape(1, H))
    y3 = spmm32(hws3, lists, cnt)
    return _pool_head(y3, dinv_col, batch2d, b3.reshape(1, H // 2),
                      Wc1, bc1.reshape(1, H), Wc2,
                      bc2.reshape(1, NUM_CLASSES))


# speedup vs baseline: 8.8556x; 2.1982x over previous
import functools

import jax
import jax.numpy as jnp
from jax import lax
from jax.experimental import pallas as pl
from jax.experimental.pallas import tpu as pltpu
from jax.experimental.pallas import tpu_sc as plsc

N = 10000
E = 320000
F_IN = 128
H = 64
G = 16
NUM_CLASSES = 2

NC = 2
NS = 16
NT = NC * NS
RPT = 320
NPAD = NT * RPT
CAP = 24576
BATCH = 128
CHUNK = 6400
TRASH = RPT
_BIS = 5


def _mesh():
    return plsc.VectorSubcoreMesh(core_axis_name="c", subcore_axis_name="s")


def _wid():
    return lax.axis_index("s") * NC + lax.axis_index("c")


def _rsqrt_newton(x):
    i = plsc.bitcast(x, jnp.int32)
    i = jnp.int32(0x5F3759DF) - lax.shift_right_arithmetic(i, 1)
    y = plsc.bitcast(i, jnp.float32)
    for _ in range(3):
        y = y * (1.5 - 0.5 * x * y * y)
    return y


NBATCH = CAP // BATCH


def _build_prep():
    nchunks = E // CHUNK

    def body(edge_hbm, list_hbm, cnt_hbm, dinv_hbm,
             ebuf, l2, deg, dinvb, cntb, sem0, sem1):
        wid = _wid()
        lo = wid * RPT
        iota = lax.iota(jnp.int32, 16)
        ones = jnp.ones((16,), jnp.float32)
        zeros_i = jnp.zeros((16,), jnp.int32)
        ones_i = jnp.ones((16,), jnp.int32)
        sems = [sem0, sem1]

        for g in range(RPT // 16):
            deg[pl.ds(g * 16, 16)] = jnp.zeros((16,), jnp.float32)

        def issue(c, s):
            pltpu.async_copy(edge_hbm.at[:, pl.ds(c * CHUNK, CHUNK)],
                             ebuf.at[s], sems[s])

        def wait(s):
            pltpu.make_async_copy(edge_hbm.at[:, pl.ds(0, CHUNK)],
                                  ebuf.at[s], sems[s]).wait()

        issue(0, 0)
        issue(1, 1)

        def scat(pos, s16, dl16, m):
            hi7 = lax.shift_right_logical(pos, 7)
            lo7 = jnp.bitwise_and(pos, 127)
            plsc.store_scatter(l2, [hi7, zeros_i, lo7], s16, mask=m)
            plsc.store_scatter(l2, [hi7, ones_i, lo7], dl16, mask=m)

        cntv = jnp.zeros((16,), jnp.int32)
        for c in range(nchunks):
            s = c % 2
            wait(s)

            def grp(g, cv):
                s16 = ebuf[s, 0, pl.ds(g * 16, 16)]
                d16 = ebuf[s, 1, pl.ds(g * 16, 16)]
                dl = d16 - lo
                m = (dl >= 0) & (dl < RPT)
                pos = cv + plsc.cumsum(m.astype(jnp.int32)) - 1
                scat(pos, s16, dl, m)
                plsc.addupdate_scatter(deg, [dl], ones, mask=m)
                return cv + plsc.all_reduce_population_count(m)

            cntv = lax.fori_loop(0, CHUNK // 16, grp, cntv)
            cntv = jnp.minimum(cntv, jnp.int32(CAP - CHUNK - 5 * BATCH))
            if c + 2 < nchunks:
                issue(c + 2, s)

        for g in range(4 * BATCH // 16):
            pos = cntv + (g * 16 + iota)
            scat(pos, zeros_i, jnp.full((16,), TRASH, jnp.int32),
                 jnp.ones((16,), jnp.bool_))
        cntp = jnp.bitwise_and(cntv + (2 * BATCH - 1), jnp.int32(-2 * BATCH))
        cntb[...] = cntp

        for g in range(RPT // 16):
            node = lo + g * 16 + iota
            dg = deg[pl.ds(g * 16, 16)] + 1.0
            y = _rsqrt_newton(dg)
            dinvb[pl.ds(g * 16, 16)] = jnp.where(node < N, y, 0.0)

        pltpu.sync_copy(dinvb, dinv_hbm.at[pl.ds(lo, RPT)])
        pltpu.sync_copy(cntb, cnt_hbm.at[wid])
        pltpu.sync_copy(l2, list_hbm.at[wid])

    return pl.kernel(
        body,
        out_type=(
            jax.ShapeDtypeStruct((NT, NBATCH, 2, BATCH), jnp.int32),
            jax.ShapeDtypeStruct((NT, 16), jnp.int32),
            jax.ShapeDtypeStruct((NPAD,), jnp.float32),
        ),
        mesh=_mesh(),
        compiler_params=pltpu.CompilerParams(needs_layout_passes=False,
                                             use_tc_tiling_on_sc=False),
        scratch_types=[
            pltpu.VMEM((2, 2, CHUNK), jnp.int32),
            pltpu.VMEM((NBATCH, 2, BATCH), jnp.int32),
            pltpu.VMEM((RPT,), jnp.float32),
            pltpu.VMEM((RPT,), jnp.float32),
            pltpu.VMEM((16,), jnp.int32),
            pltpu.SemaphoreType.DMA,
            pltpu.SemaphoreType.DMA,
        ],
    )


def _build_spmm(width):
    def body(hws_hbm, list_hbm, cnt_hbm, out_hbm,
             idxb, rows, acc, cntb, semi0, semi1, semg0, semg1):
        wid = _wid()
        lo = wid * RPT
        iota = lax.iota(jnp.int32, 16)
        semi = [semi0, semi1]
        semg = [semg0, semg1]

        pltpu.sync_copy(hws_hbm.at[pl.ds(lo, RPT)], acc.at[pl.ds(0, RPT)])
        zf = jnp.zeros((16,), jnp.float32)
        trow = jnp.full((16,), TRASH, jnp.int32)
        for j in range(width // 16):
            plsc.store_scatter(acc, [trow, j * 16 + iota], zf)

        pltpu.sync_copy(cnt_hbm.at[wid], cntb)
        nb2 = jnp.max(cntb[...]) // (2 * BATCH)

        def issue_idx(b, s):
            pltpu.async_copy(list_hbm.at[wid, b], idxb.at[s], semi[s])

        def wait_idx(s):
            pltpu.make_async_copy(list_hbm.at[wid, 0], idxb.at[s],
                                  semi[s]).wait()

        def issue_gather(s):
            pltpu.async_copy(hws_hbm.at[idxb.at[s, 0]], rows.at[s], semg[s])

        def wait_gather(s):
            pltpu.make_async_copy(hws_hbm.at[pl.ds(0, BATCH)], rows.at[s],
                                  semg[s]).wait()

        rids = [g * 16 + iota for g in range(BATCH // 16)]

        def accumulate(s):
            d16s = [idxb[s, 1, pl.ds(g * 16, 16)]
                    for g in range(BATCH // 16)]

            def jstep(j, js):
                cidx = jnp.bitwise_and(js + iota, width - 1)
                for g in range(BATCH // 16):
                    col = plsc.load_gather(rows.at[s], [rids[g], cidx])
                    plsc.addupdate_scatter(acc, [d16s[g], cidx], col)
                return js + 1

            lax.fori_loop(0, width, jstep, jnp.zeros((16,), jnp.int32))

        issue_idx(0, 0)
        wait_idx(0)
        issue_gather(0)
        issue_idx(1, 1)

        def pair_step(i, carry):
            for q in (0, 1):
                b = 2 * i + q
                other = 1 - q
                wait_idx(other)
                issue_gather(other)
                wait_gather(q)
                accumulate(q)
                issue_idx(b + 2, q)
            return carry

        lax.fori_loop(0, nb2, pair_step, jnp.int32(0))
        wait_gather(0)
        wait_idx(1)

        pltpu.sync_copy(acc.at[pl.ds(0, RPT)], out_hbm.at[pl.ds(lo, RPT)])

    return pl.kernel(
        body,
        out_type=jax.ShapeDtypeStruct((NPAD, width), jnp.float32),
        mesh=_mesh(),
        compiler_params=pltpu.CompilerParams(needs_layout_passes=False,
                                             use_tc_tiling_on_sc=False),
        scratch_types=[
            pltpu.VMEM((2, 2, BATCH), jnp.int32),
            pltpu.VMEM((2, BATCH, width), jnp.float32),
            pltpu.VMEM((RPT + 1, width), jnp.float32),
            pltpu.VMEM((16,), jnp.int32),
            pltpu.SemaphoreType.DMA,
            pltpu.SemaphoreType.DMA,
            pltpu.SemaphoreType.DMA,
            pltpu.SemaphoreType.DMA,
        ],
    )


_BLK = 512


def _mm_first(x, w, dinv_col):
    def body(x_ref, w_ref, dv_ref, o_ref):
        o_ref[...] = jnp.dot(x_ref[...], w_ref[...],
                             preferred_element_type=jnp.float32) * dv_ref[...]

    kin, kout = w.shape
    return pl.pallas_call(
        body,
        grid=(NPAD // _BLK,),
        in_specs=[
            pl.BlockSpec((_BLK, kin), lambda i: (i, 0)),
            pl.BlockSpec((kin, kout), lambda i: (0, 0)),
            pl.BlockSpec((_BLK, 1), lambda i: (i, 0)),
        ],
        out_specs=pl.BlockSpec((_BLK, kout), lambda i: (i, 0)),
        out_shape=jax.ShapeDtypeStruct((NPAD, kout), jnp.float32),
    )(x, w, dinv_col)


def _mm_mid(y, w, dinv_col, b_row):
    def body(y_ref, w_ref, dv_ref, b_ref, o_ref):
        h = jnp.maximum(y_ref[...] * dv_ref[...] + b_ref[...], 0.0)
        o_ref[...] = jnp.dot(h, w_ref[...],
                             preferred_element_type=jnp.float32) * dv_ref[...]

    kin, kout = w.shape
    return pl.pallas_call(
        body,
        grid=(NPAD // _BLK,),
        in_specs=[
            pl.BlockSpec((_BLK, kin), lambda i: (i, 0)),
            pl.BlockSpec((kin, kout), lambda i: (0, 0)),
            pl.BlockSpec((_BLK, 1), lambda i: (i, 0)),
            pl.BlockSpec((1, kin), lambda i: (0, 0)),
        ],
        out_specs=pl.BlockSpec((_BLK, kout), lambda i: (i, 0)),
        out_shape=jax.ShapeDtypeStruct((NPAD, kout), jnp.float32),
    )(y, w, dinv_col, b_row)


_BLKF = 1024


def _pool_head(y3, dinv_col, batch2d, b3_row, wc1, bc1_row, wc2, bc2_row):
    w3 = y3.shape[1]
    nblk = NPAD // _BLKF

    def body(y_ref, dv_ref, bat_ref, b3_ref, wc1_ref, bc1_ref, wc2_ref,
             bc2_ref, o_ref, psum, pmax):
        i = pl.program_id(0)

        @pl.when(i == 0)
        def _():
            psum[...] = jnp.zeros((G, w3), jnp.float32)
            pmax[...] = jnp.full((G, w3), -jnp.inf, jnp.float32)

        h = y_ref[...] * dv_ref[...]
        bat = bat_ref[...]
        rowi = (lax.broadcasted_iota(jnp.int32, (_BLKF, w3), 0) + i * _BLKF)
        starts = [jnp.sum((bat < g).astype(jnp.int32)) for g in range(G + 1)]
        for g in range(G):
            sg, eg = starts[g], starts[g + 1]
            m = (rowi >= sg) & (rowi < eg)
            sumg = jnp.sum(jnp.where(m, h, 0.0), axis=0, keepdims=True)
            maxg = jnp.max(jnp.where(m, h, -jnp.inf), axis=0, keepdims=True)
            psum[pl.ds(g, 1), :] += sumg
            pmax[pl.ds(g, 1), :] = jnp.maximum(pmax[pl.ds(g, 1), :], maxg)

        @pl.when(i == nblk - 1)
        def _():
            b3 = b3_ref[...]
            cnts = jnp.concatenate(
                [jnp.reshape((starts[g + 1] - starts[g]).astype(jnp.float32),
                             (1, 1)) for g in range(G)], axis=0)
            means = (psum[...] + cnts * b3) / jnp.maximum(cnts, 1.0)
            maxs = pmax[...] + b3
            z = jnp.concatenate([means, maxs], axis=1)
            r = jnp.maximum(jnp.dot(z, wc1_ref[...],
                                    preferred_element_type=jnp.float32)
                            + bc1_ref[...], 0.0)
            o_ref[...] = jnp.dot(r, wc2_ref[...],
                                 preferred_element_type=jnp.float32) \
                + bc2_ref[...]

    return pl.pallas_call(
        body,
        grid=(nblk,),
        in_specs=[
            pl.BlockSpec((_BLKF, w3), lambda i: (i, 0)),
            pl.BlockSpec((_BLKF, 1), lambda i: (i, 0)),
            pl.BlockSpec(batch2d.shape, lambda i: (0, 0)),
            pl.BlockSpec(b3_row.shape, lambda i: (0, 0)),
            pl.BlockSpec(wc1.shape, lambda i: (0, 0)),
            pl.BlockSpec(bc1_row.shape, lambda i: (0, 0)),
            pl.BlockSpec(wc2.shape, lambda i: (0, 0)),
            pl.BlockSpec(bc2_row.shape, lambda i: (0, 0)),
        ],
        out_specs=pl.BlockSpec((G, NUM_CLASSES), lambda i: (0, 0)),
        out_shape=jax.ShapeDtypeStruct((G, NUM_CLASSES), jnp.float32),
        scratch_shapes=[
            pltpu.VMEM((G, w3), jnp.float32),
            pltpu.VMEM((G, w3), jnp.float32),
        ],
    )(y3, dinv_col, batch2d, b3_row, wc1, bc1_row, wc2, bc2_row)


def kernel(x, edge_index, batch, W1, b1, W2, b2, W3, b3, Wc1, bc1, Wc2, bc2):
    prep = _build_prep()
    spmm64 = _build_spmm(H)
    spmm32 = _build_spmm(H // 2)

    lists, cnt, dinv = prep(edge_index)
    dinv_col = dinv.reshape(NPAD, 1)

    x_pad = jnp.pad(x, ((0, NPAD - N), (0, 0)))
    batch2d = jnp.pad(batch, (0, NPAD - N), constant_values=G).reshape(
        NPAD // 128, 128)

    hws1 = _mm_first(x_pad, W1, dinv_col)
    y1 = spmm64(hws1, lists, cnt)
    hws2 = _mm_mid(y1, W2, dinv_col, b1.reshape(1, H))
    y2 = spmm64(hws2, lists, cnt)
    hws3 = _mm_mid(y2, W3, dinv_col, b2.reshape(1, H))
    y3 = spmm32(hws3, lists, cnt)
    return _pool_head(y3, dinv_col, batch2d, b3.reshape(1, H // 2),
                      Wc1, bc1.reshape(1, H), Wc2,
                      bc2.reshape(1, NUM_CLASSES))


# speedup vs baseline: 9.0050x; 1.0169x over previous
import functools

import jax
import jax.numpy as jnp
from jax import lax
from jax.experimental import pallas as pl
from jax.experimental.pallas import tpu as pltpu
from jax.experimental.pallas import tpu_sc as plsc

N = 10000
E = 320000
F_IN = 128
H = 64
G = 16
NUM_CLASSES = 2

NC = 2
NS = 16
NT = NC * NS
RPT = 320
NPAD = NT * RPT
CAP = 24576
BATCH = 128
CHUNK = 6400
TRASH = RPT
_BIS = 5


def _mesh():
    return plsc.VectorSubcoreMesh(core_axis_name="c", subcore_axis_name="s")


def _wid():
    return lax.axis_index("s") * NC + lax.axis_index("c")


def _rsqrt_newton(x):
    i = plsc.bitcast(x, jnp.int32)
    i = jnp.int32(0x5F3759DF) - lax.shift_right_arithmetic(i, 1)
    y = plsc.bitcast(i, jnp.float32)
    for _ in range(3):
        y = y * (1.5 - 0.5 * x * y * y)
    return y


NBATCH = CAP // BATCH


def _build_prep():
    nchunks = E // CHUNK

    def body(edge_hbm, list_hbm, cnt_hbm, dinv_hbm,
             ebuf, l2, deg, dinvb, cntb, sem0, sem1):
        wid = _wid()
        lo = wid * RPT
        iota = lax.iota(jnp.int32, 16)
        ones = jnp.ones((16,), jnp.float32)
        zeros_i = jnp.zeros((16,), jnp.int32)
        ones_i = jnp.ones((16,), jnp.int32)
        sems = [sem0, sem1]

        for g in range(RPT // 16):
            deg[pl.ds(g * 16, 16)] = jnp.zeros((16,), jnp.float32)

        def issue(c, s):
            pltpu.async_copy(edge_hbm.at[:, pl.ds(c * CHUNK, CHUNK)],
                             ebuf.at[s], sems[s])

        def wait(s):
            pltpu.make_async_copy(edge_hbm.at[:, pl.ds(0, CHUNK)],
                                  ebuf.at[s], sems[s]).wait()

        issue(0, 0)
        issue(1, 1)

        def scat(pos, s16, dl16, m):
            hi7 = lax.shift_right_logical(pos, 7)
            lo7 = jnp.bitwise_and(pos, 127)
            plsc.store_scatter(l2, [hi7, zeros_i, lo7], s16, mask=m)
            plsc.store_scatter(l2, [hi7, ones_i, lo7], dl16, mask=m)

        cntv = jnp.zeros((16,), jnp.int32)
        for c in range(nchunks):
            s = c % 2
            wait(s)

            def grp(g, cv):
                for u in range(4):
                    s16 = ebuf[s, 0, pl.ds((g * 4 + u) * 16, 16)]
                    d16 = ebuf[s, 1, pl.ds((g * 4 + u) * 16, 16)]
                    dl = d16 - lo
                    m = (dl >= 0) & (dl < RPT)
                    pos = cv + plsc.cumsum(m.astype(jnp.int32)) - 1
                    scat(pos, s16, dl, m)
                    plsc.addupdate_scatter(deg, [dl], ones, mask=m)
                    cv = cv + plsc.all_reduce_population_count(m)
                return cv

            cntv = lax.fori_loop(0, CHUNK // 64, grp, cntv)
            cntv = jnp.minimum(cntv, jnp.int32(CAP - CHUNK - 5 * BATCH))
            if c + 2 < nchunks:
                issue(c + 2, s)

        for g in range(4 * BATCH // 16):
            pos = cntv + (g * 16 + iota)
            scat(pos, zeros_i, jnp.full((16,), TRASH, jnp.int32),
                 jnp.ones((16,), jnp.bool_))
        cntp = jnp.bitwise_and(cntv + (2 * BATCH - 1), jnp.int32(-2 * BATCH))
        cntb[...] = cntp

        for g in range(RPT // 16):
            node = lo + g * 16 + iota
            dg = deg[pl.ds(g * 16, 16)] + 1.0
            y = _rsqrt_newton(dg)
            dinvb[pl.ds(g * 16, 16)] = jnp.where(node < N, y, 0.0)

        pltpu.sync_copy(dinvb, dinv_hbm.at[pl.ds(lo, RPT)])
        pltpu.sync_copy(cntb, cnt_hbm.at[wid])
        pltpu.sync_copy(l2, list_hbm.at[wid])

    return pl.kernel(
        body,
        out_type=(
            jax.ShapeDtypeStruct((NT, NBATCH, 2, BATCH), jnp.int32),
            jax.ShapeDtypeStruct((NT, 16), jnp.int32),
            jax.ShapeDtypeStruct((NPAD,), jnp.float32),
        ),
        mesh=_mesh(),
        compiler_params=pltpu.CompilerParams(needs_layout_passes=False,
                                             use_tc_tiling_on_sc=False),
        scratch_types=[
            pltpu.VMEM((2, 2, CHUNK), jnp.int32),
            pltpu.VMEM((NBATCH, 2, BATCH), jnp.int32),
            pltpu.VMEM((RPT,), jnp.float32),
            pltpu.VMEM((RPT,), jnp.float32),
            pltpu.VMEM((16,), jnp.int32),
            pltpu.SemaphoreType.DMA,
            pltpu.SemaphoreType.DMA,
        ],
    )


def _build_spmm(width):
    def body(hws_hbm, list_hbm, cnt_hbm, out_hbm,
             idxb, rows, acc, cntb, semi0, semi1, semg0, semg1):
        wid = _wid()
        lo = wid * RPT
        iota = lax.iota(jnp.int32, 16)
        semi = [semi0, semi1]
        semg = [semg0, semg1]

        pltpu.sync_copy(hws_hbm.at[pl.ds(lo, RPT)], acc.at[pl.ds(0, RPT)])
        zf = jnp.zeros((16,), jnp.float32)
        trow = jnp.full((16,), TRASH, jnp.int32)
        for j in range(width // 16):
            plsc.store_scatter(acc, [trow, j * 16 + iota], zf)

        pltpu.sync_copy(cnt_hbm.at[wid], cntb)
        nb2 = jnp.max(cntb[...]) // (2 * BATCH)

        def issue_idx(b, s):
            pltpu.async_copy(list_hbm.at[wid, b], idxb.at[s], semi[s])

        def wait_idx(s):
            pltpu.make_async_copy(list_hbm.at[wid, 0], idxb.at[s],
                                  semi[s]).wait()

        def issue_gather(s):
            pltpu.async_copy(hws_hbm.at[idxb.at[s, 0]], rows.at[s], semg[s])

        def wait_gather(s):
            pltpu.make_async_copy(hws_hbm.at[pl.ds(0, BATCH)], rows.at[s],
                                  semg[s]).wait()

        rids = [g * 16 + iota for g in range(BATCH // 16)]

        def accumulate(s):
            d16s = [idxb[s, 1, pl.ds(g * 16, 16)]
                    for g in range(BATCH // 16)]

            def jstep(j, js):
                for u in range(2):
                    cidx = jnp.bitwise_and(js + (iota + u), width - 1)
                    for g in range(BATCH // 16):
                        col = plsc.load_gather(rows.at[s], [rids[g], cidx])
                        plsc.addupdate_scatter(acc, [d16s[g], cidx], col)
                return js + 2

            lax.fori_loop(0, width // 2, jstep, jnp.zeros((16,), jnp.int32))

        issue_idx(0, 0)
        wait_idx(0)
        issue_gather(0)
        issue_idx(1, 1)

        def pair_step(i, carry):
            for q in (0, 1):
                b = 2 * i + q
                other = 1 - q
                wait_idx(other)
                issue_gather(other)
                wait_gather(q)
                accumulate(q)
                issue_idx(b + 2, q)
            return carry

        lax.fori_loop(0, nb2, pair_step, jnp.int32(0))
        wait_gather(0)
        wait_idx(1)

        pltpu.sync_copy(acc.at[pl.ds(0, RPT)], out_hbm.at[pl.ds(lo, RPT)])

    return pl.kernel(
        body,
        out_type=jax.ShapeDtypeStruct((NPAD, width), jnp.float32),
        mesh=_mesh(),
        compiler_params=pltpu.CompilerParams(needs_layout_passes=False,
                                             use_tc_tiling_on_sc=False),
        scratch_types=[
            pltpu.VMEM((2, 2, BATCH), jnp.int32),
            pltpu.VMEM((2, BATCH, width), jnp.float32),
            pltpu.VMEM((RPT + 1, width), jnp.float32),
            pltpu.VMEM((16,), jnp.int32),
            pltpu.SemaphoreType.DMA,
            pltpu.SemaphoreType.DMA,
            pltpu.SemaphoreType.DMA,
            pltpu.SemaphoreType.DMA,
        ],
    )


_BLK = 512


def _mm_first(x, w, dinv_col):
    def body(x_ref, w_ref, dv_ref, o_ref):
        o_ref[...] = jnp.dot(x_ref[...], w_ref[...],
                             preferred_element_type=jnp.float32) * dv_ref[...]

    kin, kout = w.shape
    return pl.pallas_call(
        body,
        grid=(NPAD // _BLK,),
        in_specs=[
            pl.BlockSpec((_BLK, kin), lambda i: (i, 0)),
            pl.BlockSpec((kin, kout), lambda i: (0, 0)),
            pl.BlockSpec((_BLK, 1), lambda i: (i, 0)),
        ],
        out_specs=pl.BlockSpec((_BLK, kout), lambda i: (i, 0)),
        out_shape=jax.ShapeDtypeStruct((NPAD, kout), jnp.float32),
    )(x, w, dinv_col)


def _mm_mid(y, w, dinv_col, b_row):
    def body(y_ref, w_ref, dv_ref, b_ref, o_ref):
        h = jnp.maximum(y_ref[...] * dv_ref[...] + b_ref[...], 0.0)
        o_ref[...] = jnp.dot(h, w_ref[...],
                             preferred_element_type=jnp.float32) * dv_ref[...]

    kin, kout = w.shape
    return pl.pallas_call(
        body,
        grid=(NPAD // _BLK,),
        in_specs=[
            pl.BlockSpec((_BLK, kin), lambda i: (i, 0)),
            pl.BlockSpec((kin, kout), lambda i: (0, 0)),
            pl.BlockSpec((_BLK, 1), lambda i: (i, 0)),
            pl.BlockSpec((1, kin), lambda i: (0, 0)),
        ],
        out_specs=pl.BlockSpec((_BLK, kout), lambda i: (i, 0)),
        out_shape=jax.ShapeDtypeStruct((NPAD, kout), jnp.float32),
    )(y, w, dinv_col, b_row)


_BLKF = 1024


def _pool_head(y3, dinv_col, batch2d, b3_row, wc1, bc1_row, wc2, bc2_row):
    w3 = y3.shape[1]
    nblk = NPAD // _BLKF

    def body(y_ref, dv_ref, bat_ref, b3_ref, wc1_ref, bc1_ref, wc2_ref,
             bc2_ref, o_ref, psum, pmax):
        i = pl.program_id(0)

        @pl.when(i == 0)
        def _():
            psum[...] = jnp.zeros((G, w3), jnp.float32)
            pmax[...] = jnp.full((G, w3), -jnp.inf, jnp.float32)

        h = y_ref[...] * dv_ref[...]
        bat = bat_ref[...]
        rowi = (lax.broadcasted_iota(jnp.int32, (_BLKF, w3), 0) + i * _BLKF)
        starts = [jnp.sum((bat < g).astype(jnp.int32)) for g in range(G + 1)]
        for g in range(G):
            sg, eg = starts[g], starts[g + 1]
            m = (rowi >= sg) & (rowi < eg)
            sumg = jnp.sum(jnp.where(m, h, 0.0), axis=0, keepdims=True)
            maxg = jnp.max(jnp.where(m, h, -jnp.inf), axis=0, keepdims=True)
            psum[pl.ds(g, 1), :] += sumg
            pmax[pl.ds(g, 1), :] = jnp.maximum(pmax[pl.ds(g, 1), :], maxg)

        @pl.when(i == nblk - 1)
        def _():
            b3 = b3_ref[...]
            cnts = jnp.concatenate(
                [jnp.reshape((starts[g + 1] - starts[g]).astype(jnp.float32),
                             (1, 1)) for g in range(G)], axis=0)
            means = (psum[...] + cnts * b3) / jnp.maximum(cnts, 1.0)
            maxs = pmax[...] + b3
            z = jnp.concatenate([means, maxs], axis=1)
            r = jnp.maximum(jnp.dot(z, wc1_ref[...],
                                    preferred_element_type=jnp.float32)
                            + bc1_ref[...], 0.0)
            o_ref[...] = jnp.dot(r, wc2_ref[...],
                                 preferred_element_type=jnp.float32) \
                + bc2_ref[...]

    return pl.pallas_call(
        body,
        grid=(nblk,),
        in_specs=[
            pl.BlockSpec((_BLKF, w3), lambda i: (i, 0)),
            pl.BlockSpec((_BLKF, 1), lambda i: (i, 0)),
            pl.BlockSpec(batch2d.shape, lambda i: (0, 0)),
            pl.BlockSpec(b3_row.shape, lambda i: (0, 0)),
            pl.BlockSpec(wc1.shape, lambda i: (0, 0)),
            pl.BlockSpec(bc1_row.shape, lambda i: (0, 0)),
            pl.BlockSpec(wc2.shape, lambda i: (0, 0)),
            pl.BlockSpec(bc2_row.shape, lambda i: (0, 0)),
        ],
        out_specs=pl.BlockSpec((G, NUM_CLASSES), lambda i: (0, 0)),
        out_shape=jax.ShapeDtypeStruct((G, NUM_CLASSES), jnp.float32),
        scratch_shapes=[
            pltpu.VMEM((G, w3), jnp.float32),
            pltpu.VMEM((G, w3), jnp.float32),
        ],
    )(y3, dinv_col, batch2d, b3_row, wc1, bc1_row, wc2, bc2_row)


def kernel(x, edge_index, batch, W1, b1, W2, b2, W3, b3, Wc1, bc1, Wc2, bc2):
    prep = _build_prep()
    spmm64 = _build_spmm(H)
    spmm32 = _build_spmm(H // 2)

    lists, cnt, dinv = prep(edge_index)
    dinv_col = dinv.reshape(NPAD, 1)

    x_pad = jnp.pad(x, ((0, NPAD - N), (0, 0)))
    batch2d = jnp.pad(batch, (0, NPAD - N), constant_values=G).reshape(
        NPAD // 128, 128)

    hws1 = _mm_first(x_pad, W1, dinv_col)
    y1 = spmm64(hws1, lists, cnt)
    hws2 = _mm_mid(y1, W2, dinv_col, b1.reshape(1, H))
    y2 = spmm64(hws2, lists, cnt)
    hws3 = _mm_mid(y2, W3, dinv_col, b2.reshape(1, H))
    y3 = spmm32(hws3, lists, cnt)
    return _pool_head(y3, dinv_col, batch2d, b3.reshape(1, H // 2),
                      Wc1, bc1.reshape(1, H), Wc2,
                      bc2.reshape(1, NUM_CLASSES))
